# baseline (speedup 1.0000x reference)
import jax
import jax.numpy as jnp
from jax import lax
from jax.experimental import pallas as pl
from jax.experimental.pallas import tpu as pltpu
from jax.experimental.pallas import tpu_sc as plsc

NUM_WORDS_K = 1000000
NUM_BUCKETS_K = 100000
EMB_K = 64
BATCH_K = 4096
SEQ_K = 200

NC = 2
NS = 16
LANES = 16
NW = NC * NS

N_TOK = BATCH_K * SEQ_K
TOK_PER_W = N_TOK // NW
CHUNK = 256
N_CHUNKS = TOK_PER_W // CHUNK
OUT_COLS = EMB_K + 2
WORDS_PAD = 3 * CHUNK


def _sc_body(words, ht0, ht1, p0c, p1c, w_tab, out,
             wid_a, wid_b, h0_a, h0_b, h1_a, h1_b, pa_v, pb_v,
             w0_a, w0_b, w1_a, w1_b, pv0_v, pv1_v, out_a, out_b,
             sw0, sw1, s10, s11, s20, s21, so0, so1):
    wid = lax.axis_index("s") * NC + lax.axis_index("c")
    base_w = wid * TOK_PER_W

    iota = lax.iota(jnp.int32, LANES)
    wid_bufs = (wid_a, wid_b)
    h0_bufs = (h0_a, h0_b)
    h1_bufs = (h1_a, h1_b)
    w0_bufs = (w0_a, w0_b)
    w1_bufs = (w1_a, w1_b)
    out_bufs = (out_a, out_b)
    sem_w = (sw0, sw1)
    sem_1 = (s10, s11)
    sem_2 = (s20, s21)
    sem_o = (so0, so1)

    def words_copy(g, b):
        return pltpu.make_async_copy(
            words.at[pl.ds(base_w + g * CHUNK, CHUNK)], wid_bufs[b], sem_w[b])

    def small_copies(b, q):
        return [
            pltpu.make_async_copy(ht0.at[wid_bufs[b]], h0_bufs[b], sem_1[b]),
            pltpu.make_async_copy(ht1.at[wid_bufs[b]], h1_bufs[b], sem_1[b]),
            pltpu.make_async_copy(p0c.at[wid_bufs[b]],
                                  pa_v.at[pl.ds(q * CHUNK, CHUNK)], sem_1[b]),
            pltpu.make_async_copy(p1c.at[wid_bufs[b]],
                                  pb_v.at[pl.ds(q * CHUNK, CHUNK)], sem_1[b]),
        ]

    def big_copies(b):
        return [
            pltpu.make_async_copy(w_tab.at[h0_bufs[b]], w0_bufs[b], sem_2[b]),
            pltpu.make_async_copy(w_tab.at[h1_bufs[b]], w1_bufs[b], sem_2[b]),
            pltpu.make_async_copy(p0c.at[h0_bufs[b]],
                                  pv0_v.at[pl.ds(b * CHUNK, CHUNK)], sem_2[b]),
            pltpu.make_async_copy(p1c.at[h1_bufs[b]],
                                  pv1_v.at[pl.ds(b * CHUNK, CHUNK)], sem_2[b]),
        ]

    def out_copy(g, b):
        return pltpu.make_async_copy(
            out_bufs[b],
            out.at[pl.ds((base_w + g * CHUNK) * OUT_COLS, CHUNK * OUT_COLS)],
            sem_o[b])

    def compute(g, b, q):
        out_v = out_bufs[b]
        w0_v = w0_bufs[b]
        w1_v = w1_bufs[b]
        poff = q * CHUNK
        voff = b * CHUNK

        iota2 = iota * 2

        @plsc.parallel_loop(0, CHUNK, 1, unroll=4)
        def tok_body(t):
            tsplat = jnp.full((LANES,), t + poff, jnp.int32)
            p0 = plsc.load_gather(pa_v, [tsplat])
            p1 = plsc.load_gather(pb_v, [tsplat])
            obase = t * OUT_COLS
            for k2 in range(EMB_K // 32):
                a0 = w0_v[t, pl.ds(32 * k2, 32)]
                a1 = w1_v[t, pl.ds(32 * k2, 32)]
                e0, o0 = plsc.unpack(a0, format=plsc.PackFormat.INTERLEAVED)
                e1, o1 = plsc.unpack(a1, format=plsc.PackFormat.INTERLEAVED)
                b2 = obase + 32 * k2
                plsc.store_scatter(out_v, [b2 + iota2], e0 * p0 + e1 * p1)
                plsc.store_scatter(out_v, [b2 + iota2 + 1], o0 * p0 + o1 * p1)

        @plsc.parallel_loop(0, CHUNK // LANES, 1, unroll=2)
        def pv_body(g16):
            rows = iota + g16 * LANES
            obase = rows * OUT_COLS
            pv0 = plsc.load_gather(pv0_v, [rows + voff])
            pv1 = plsc.load_gather(pv1_v, [rows + voff])
            plsc.store_scatter(out_v, [obase + EMB_K], pv0)
            plsc.store_scatter(out_v, [obase + EMB_K + 1], pv1)

    def slot(g, b, q, *, out_wait=True, issue_w=True, issue_small=True,
             issue_words=True, words_wait=True):
        for c in big_copies(b):
            c.wait()
        if words_wait:
            words_copy(g + 2, b).wait()
        if issue_w:
            for c in small_copies(b ^ 1, (q + 1) % 4):
                c.wait()
            for c in big_copies(b ^ 1):
                c.start()
        if issue_small:
            for c in small_copies(b, (q + 2) % 4):
                c.start()
        if issue_words:
            words_copy(g + 3, b ^ 1).start()
        if out_wait:
            out_copy(g - 2, b).wait()
        compute(g, b, q)
        out_copy(g, b).start()

    words_copy(0, 0).start()
    words_copy(1, 1).start()
    words_copy(0, 0).wait()
    for c in small_copies(0, 0):
        c.start()
    words_copy(1, 1).wait()
    for c in small_copies(1, 1):
        c.start()
    for c in small_copies(0, 0):
        c.wait()
    for c in big_copies(0):
        c.start()
    words_copy(2, 0).start()

    slot(jnp.int32(0), 0, 0, out_wait=False)
    slot(jnp.int32(1), 1, 1, out_wait=False)

    def pair_body(p, carry):
        g0 = 2 + 4 * p
        slot(g0, 0, 2)
        slot(g0 + 1, 1, 3)
        slot(g0 + 2, 0, 0)
        slot(g0 + 3, 1, 1)
        return carry

    lax.fori_loop(0, (N_CHUNKS - 8) // 4, pair_body, None)

    g0 = jnp.int32(N_CHUNKS - 6)
    slot(g0, 0, 2)
    slot(g0 + 1, 1, 3)
    slot(g0 + 2, 0, 0)
    slot(g0 + 3, 1, 1)
    slot(g0 + 4, 0, 2, issue_small=False, issue_words=False)
    slot(g0 + 5, 1, 3, issue_w=False, issue_small=False, issue_words=False,
         words_wait=False)
    out_copy(g0 + 4, 0).wait()
    out_copy(g0 + 5, 1).wait()


@jax.jit
def kernel(words_as_ids, hash_table, W, P):
    words_flat = jnp.concatenate(
        [words_as_ids.reshape(N_TOK).astype(jnp.int32),
         jnp.zeros((WORDS_PAD,), jnp.int32)])
    ht0 = hash_table[:, 0]
    ht1 = hash_table[:, 1]
    p0c = P[:, 0]
    p1c = P[:, 1]

    mesh = plsc.VectorSubcoreMesh(core_axis_name="c", subcore_axis_name="s",
                                  num_cores=NC, num_subcores=NS)
    out = pl.kernel(
        _sc_body,
        out_type=jax.ShapeDtypeStruct((N_TOK * OUT_COLS,), jnp.float32),
        mesh=mesh,
        compiler_params=pltpu.CompilerParams(
            needs_layout_passes=False, use_tc_tiling_on_sc=False),
        scratch_types=[
            pltpu.VMEM((CHUNK,), jnp.int32),
            pltpu.VMEM((CHUNK,), jnp.int32),
            pltpu.VMEM((CHUNK,), jnp.int32),
            pltpu.VMEM((CHUNK,), jnp.int32),
            pltpu.VMEM((CHUNK,), jnp.int32),
            pltpu.VMEM((CHUNK,), jnp.int32),
            pltpu.VMEM((4 * CHUNK,), jnp.float32),
            pltpu.VMEM((4 * CHUNK,), jnp.float32),
            pltpu.VMEM((CHUNK, EMB_K), jnp.bfloat16),
            pltpu.VMEM((CHUNK, EMB_K), jnp.bfloat16),
            pltpu.VMEM((CHUNK, EMB_K), jnp.bfloat16),
            pltpu.VMEM((CHUNK, EMB_K), jnp.bfloat16),
            pltpu.VMEM((2 * CHUNK,), jnp.float32),
            pltpu.VMEM((2 * CHUNK,), jnp.float32),
            pltpu.VMEM((CHUNK * OUT_COLS,), jnp.float32),
            pltpu.VMEM((CHUNK * OUT_COLS,), jnp.float32),
            pltpu.SemaphoreType.DMA,
            pltpu.SemaphoreType.DMA,
            pltpu.SemaphoreType.DMA,
            pltpu.SemaphoreType.DMA,
            pltpu.SemaphoreType.DMA,
            pltpu.SemaphoreType.DMA,
            pltpu.SemaphoreType.DMA,
            pltpu.SemaphoreType.DMA,
        ],
    )(words_flat, ht0, ht1, p0c, p1c, W.astype(jnp.bfloat16))
    return out.reshape(BATCH_K, SEQ_K, OUT_COLS)


# speedup vs baseline: 1.0080x; 1.0080x over previous
import jax
import jax.numpy as jnp
from jax import lax
from jax.experimental import pallas as pl
from jax.experimental.pallas import tpu as pltpu
from jax.experimental.pallas import tpu_sc as plsc

NUM_WORDS_K = 1000000
NUM_BUCKETS_K = 100000
EMB_K = 64
BATCH_K = 4096
SEQ_K = 200

NC = 2
NS = 16
LANES = 16
NW = NC * NS

N_TOK = BATCH_K * SEQ_K
TOK_PER_W = N_TOK // NW
CHUNK = 256
N_CHUNKS = TOK_PER_W // CHUNK
OUT_COLS = EMB_K + 2
WORDS_PAD = 3 * CHUNK
PADN = 1000448
PSLICE = PADN // NS
PVN = 100096
PVSLICE = PVN // NS


def _sc_body(words, ht0, ht1, ppk, pvk, w_tab, out,
             wid_a, wid_b, h0_a, h0_b, h1_a, h1_b, pp_v,
             w0_a, w0_b, w1_a, w1_b, pk0_v, pk1_v, out_a, out_b,
             pa_f, pb_f,
             sw0, sw1, s10, s11, s20, s21, so0, so1):
    sid = lax.axis_index("s")
    wid = sid * NC + lax.axis_index("c")
    base_w = wid * TOK_PER_W

    iota = lax.iota(jnp.int32, LANES)
    wid_bufs = (wid_a, wid_b)
    h0_bufs = (h0_a, h0_b)
    h1_bufs = (h1_a, h1_b)
    w0_bufs = (w0_a, w0_b)
    w1_bufs = (w1_a, w1_b)
    out_bufs = (out_a, out_b)
    sem_w = (sw0, sw1)
    sem_1 = (s10, s11)
    sem_2 = (s20, s21)
    sem_o = (so0, so1)

    def words_copy(g, b):
        return pltpu.make_async_copy(
            words.at[pl.ds(base_w + g * CHUNK, CHUNK)], wid_bufs[b], sem_w[b])

    def small_copies(b, q):
        return [
            pltpu.make_async_copy(ht0.at[wid_bufs[b]], h0_bufs[b], sem_1[b]),
            pltpu.make_async_copy(ht1.at[wid_bufs[b]], h1_bufs[b], sem_1[b]),
            pltpu.make_async_copy(ppk.at[wid_bufs[b]],
                                  pp_v.at[pl.ds(q * CHUNK, CHUNK)], sem_1[b]),
        ]

    def big_copies(b):
        return [
            pltpu.make_async_copy(w_tab.at[h0_bufs[b]], w0_bufs[b], sem_2[b]),
            pltpu.make_async_copy(w_tab.at[h1_bufs[b]], w1_bufs[b], sem_2[b]),
            pltpu.make_async_copy(pvk.at[h0_bufs[b]],
                                  pk0_v.at[pl.ds(b * CHUNK, CHUNK)], sem_2[b]),
            pltpu.make_async_copy(pvk.at[h1_bufs[b]],
                                  pk1_v.at[pl.ds(b * CHUNK, CHUNK)], sem_2[b]),
        ]

    def out_copy(g, b):
        return pltpu.make_async_copy(
            out_bufs[b],
            out.at[pl.ds((base_w + g * CHUNK) * OUT_COLS, CHUNK * OUT_COLS)],
            sem_o[b])

    def compute(g, b, q):
        out_v = out_bufs[b]
        w0_v = w0_bufs[b]
        w1_v = w1_bufs[b]
        poff = q * CHUNK
        voff = b * CHUNK

        iota2 = iota * 2

        @plsc.parallel_loop(0, CHUNK // LANES, 1, unroll=2)
        def pdec_body(k):
            v = pp_v[pl.ds(poff + k * LANES, LANES)]
            lo, hi = plsc.unpack(plsc.bitcast(v, jnp.bfloat16),
                                 format=plsc.PackFormat.INTERLEAVED)
            sl = pl.ds(k * LANES, LANES)
            pa_f[sl] = lo
            pb_f[sl] = hi

        @plsc.parallel_loop(0, CHUNK, 1, unroll=4)
        def tok_body(t):
            tsplat = jnp.full((LANES,), t, jnp.int32)
            p0 = plsc.load_gather(pa_f, [tsplat])
            p1 = plsc.load_gather(pb_f, [tsplat])
            obase = t * OUT_COLS
            for k2 in range(EMB_K // 32):
                a0 = w0_v[t, pl.ds(32 * k2, 32)]
                a1 = w1_v[t, pl.ds(32 * k2, 32)]
                e0, o0 = plsc.unpack(a0, format=plsc.PackFormat.INTERLEAVED)
                e1, o1 = plsc.unpack(a1, format=plsc.PackFormat.INTERLEAVED)
                b2 = obase + 32 * k2
                plsc.store_scatter(out_v, [b2 + iota2], e0 * p0 + e1 * p1)
                plsc.store_scatter(out_v, [b2 + iota2 + 1], o0 * p0 + o1 * p1)

        @plsc.parallel_loop(0, CHUNK // LANES, 1, unroll=2)
        def pv_body(g16):
            rows = iota + g16 * LANES
            obase = rows * OUT_COLS
            sl = pl.ds(voff + g16 * LANES, LANES)
            pv0, _ = plsc.unpack(plsc.bitcast(pk0_v[sl], jnp.bfloat16),
                                 format=plsc.PackFormat.INTERLEAVED)
            _, pv1 = plsc.unpack(plsc.bitcast(pk1_v[sl], jnp.bfloat16),
                                 format=plsc.PackFormat.INTERLEAVED)
            plsc.store_scatter(out_v, [obase + EMB_K], pv0)
            plsc.store_scatter(out_v, [obase + EMB_K + 1], pv1)

    def slot(g, b, q, *, out_wait=True, issue_w=True, issue_small=True,
             issue_words=True, words_wait=True):
        for c in big_copies(b):
            c.wait()
        if words_wait:
            words_copy(g + 2, b).wait()
        if issue_w:
            for c in small_copies(b ^ 1, (q + 1) % 4):
                c.wait()
            for c in big_copies(b ^ 1):
                c.start()
        if issue_small:
            for c in small_copies(b, (q + 2) % 4):
                c.start()
        if issue_words:
            words_copy(g + 3, b ^ 1).start()
        if out_wait:
            out_copy(g - 2, b).wait()
        compute(g, b, q)
        out_copy(g, b).start()

    words_copy(0, 0).start()
    words_copy(1, 1).start()
    words_copy(0, 0).wait()
    for c in small_copies(0, 0):
        c.start()
    words_copy(1, 1).wait()
    for c in small_copies(1, 1):
        c.start()
    for c in small_copies(0, 0):
        c.wait()
    for c in big_copies(0):
        c.start()
    words_copy(2, 0).start()

    slot(jnp.int32(0), 0, 0, out_wait=False)
    slot(jnp.int32(1), 1, 1, out_wait=False)

    def pair_body(p, carry):
        g0 = 2 + 4 * p
        slot(g0, 0, 2)
        slot(g0 + 1, 1, 3)
        slot(g0 + 2, 0, 0)
        slot(g0 + 3, 1, 1)
        return carry

    lax.fori_loop(0, (N_CHUNKS - 8) // 4, pair_body, None)

    g0 = jnp.int32(N_CHUNKS - 6)
    slot(g0, 0, 2)
    slot(g0 + 1, 1, 3)
    slot(g0 + 2, 0, 0)
    slot(g0 + 3, 1, 1)
    slot(g0 + 4, 0, 2, issue_small=False, issue_words=False)
    slot(g0 + 5, 1, 3, issue_w=False, issue_small=False, issue_words=False,
         words_wait=False)
    out_copy(g0 + 4, 0).wait()
    out_copy(g0 + 5, 1).wait()


def _pack_pairs(a, b, n_pad):
    lo = lax.bitcast_convert_type(a.astype(jnp.bfloat16), jnp.uint16)
    hi = lax.bitcast_convert_type(b.astype(jnp.bfloat16), jnp.uint16)
    packed = lo.astype(jnp.uint32) | (hi.astype(jnp.uint32) << 16)
    packed = lax.bitcast_convert_type(packed, jnp.int32)
    return jnp.concatenate([packed, jnp.zeros((n_pad,), jnp.int32)])


@jax.jit
def kernel(words_as_ids, hash_table, W, P):
    words_flat = jnp.concatenate(
        [words_as_ids.reshape(N_TOK).astype(jnp.int32),
         jnp.zeros((WORDS_PAD,), jnp.int32)])
    ht0 = hash_table[:, 0]
    ht1 = hash_table[:, 1]
    ppk = _pack_pairs(P[:, 0], P[:, 1], PADN - NUM_WORDS_K)
    pvk = _pack_pairs(P[:NUM_BUCKETS_K, 0], P[:NUM_BUCKETS_K, 1],
                      PVN - NUM_BUCKETS_K)

    mesh = plsc.VectorSubcoreMesh(core_axis_name="c", subcore_axis_name="s",
                                  num_cores=NC, num_subcores=NS)
    out = pl.kernel(
        _sc_body,
        out_type=jax.ShapeDtypeStruct((N_TOK * OUT_COLS,), jnp.float32),
        mesh=mesh,
        compiler_params=pltpu.CompilerParams(
            needs_layout_passes=False, use_tc_tiling_on_sc=False),
        scratch_types=[
            pltpu.VMEM((CHUNK,), jnp.int32),
            pltpu.VMEM((CHUNK,), jnp.int32),
            pltpu.VMEM((CHUNK,), jnp.int32),
            pltpu.VMEM((CHUNK,), jnp.int32),
            pltpu.VMEM((CHUNK,), jnp.int32),
            pltpu.VMEM((CHUNK,), jnp.int32),
            pltpu.VMEM((4 * CHUNK,), jnp.int32),
            pltpu.VMEM((CHUNK, EMB_K), jnp.bfloat16),
            pltpu.VMEM((CHUNK, EMB_K), jnp.bfloat16),
            pltpu.VMEM((CHUNK, EMB_K), jnp.bfloat16),
            pltpu.VMEM((CHUNK, EMB_K), jnp.bfloat16),
            pltpu.VMEM((2 * CHUNK,), jnp.int32),
            pltpu.VMEM((2 * CHUNK,), jnp.int32),
            pltpu.VMEM((CHUNK * OUT_COLS,), jnp.float32),
            pltpu.VMEM((CHUNK * OUT_COLS,), jnp.float32),
            pltpu.VMEM((CHUNK,), jnp.float32),
            pltpu.VMEM((CHUNK,), jnp.float32),
            pltpu.SemaphoreType.DMA,
            pltpu.SemaphoreType.DMA,
            pltpu.SemaphoreType.DMA,
            pltpu.SemaphoreType.DMA,
            pltpu.SemaphoreType.DMA,
            pltpu.SemaphoreType.DMA,
            pltpu.SemaphoreType.DMA,
            pltpu.SemaphoreType.DMA,
        ],
    )(words_flat, ht0, ht1, ppk, pvk, W.astype(jnp.bfloat16))
    return out.reshape(BATCH_K, SEQ_K, OUT_COLS)
